# baseline (speedup 1.0000x reference)
import functools

import jax
import jax.numpy as jnp
from jax import lax
from jax.experimental import pallas as pl
from jax.experimental.pallas import tpu as pltpu
from jax.experimental.pallas import tpu_sc as plsc

N, E, D, H, R = 10000, 320000, 128, 128, 500

NC, NS, L = 2, 16, 16
NW = NC * NS
EPW = E // NW
B = 80
ROUNDS = EPW // B
KJ = H // L

M_BLK = 400

_DNUMS = lax.GatherDimensionNumbers(
    offset_dims=(), collapsed_slice_dims=(0,), start_index_map=(0,))


def _vperm(vec, idx):
  return lax.gather(vec, idx[:, None], _DNUMS, (1,),
                    mode=lax.GatherScatterMode.PROMISE_IN_BOUNDS)


def _transform_body(x_ref, wh_ref, wt_ref, bh_ref, bt_ref, hh_ref, ht_ref):
    xb = x_ref[...]
    hh_ref[...] = jnp.dot(xb, wh_ref[...],
                          preferred_element_type=jnp.float32) + bh_ref[...]
    ht_ref[...] = jnp.dot(xb, wt_ref[...],
                          preferred_element_type=jnp.float32) + bt_ref[...]


_node_transform = pl.pallas_call(
    _transform_body,
    grid=(N // M_BLK,),
    in_specs=[
        pl.BlockSpec((M_BLK, D), lambda i: (i, 0)),
        pl.BlockSpec((D, H), lambda i: (0, 0)),
        pl.BlockSpec((D, H), lambda i: (0, 0)),
        pl.BlockSpec((1, H), lambda i: (0, 0)),
        pl.BlockSpec((1, H), lambda i: (0, 0)),
    ],
    out_specs=[
        pl.BlockSpec((M_BLK, H), lambda i: (i, 0)),
        pl.BlockSpec((M_BLK, H), lambda i: (i, 0)),
    ],
    out_shape=[
        jax.ShapeDtypeStruct((N, H), jnp.float32),
        jax.ShapeDtypeStruct((N, H), jnp.float32),
    ],
)


@functools.cache
def _build_edge_score():
  @functools.partial(
    pl.kernel,
    out_type=jax.ShapeDtypeStruct((E,), jnp.float32),
    mesh=plsc.VectorSubcoreMesh(core_axis_name="c", subcore_axis_name="s",
                                num_cores=NC, num_subcores=NS),
    scratch_types=[
        pltpu.VMEM((4, B), jnp.int32),
        pltpu.VMEM((4, B), jnp.int32),
        pltpu.VMEM((4, B), jnp.int32),
        pltpu.VMEM((B, H), jnp.float32),
        pltpu.VMEM((B, H), jnp.float32),
        pltpu.VMEM((B, H), jnp.float32),
        pltpu.VMEM((B, H), jnp.float32),
        pltpu.VMEM((R, H), jnp.float32),
        pltpu.VMEM((EPW,), jnp.float32),
        pltpu.VMEM((B // L, L, 17), jnp.float32),
        pltpu.SemaphoreType.DMA,
        pltpu.SemaphoreType.DMA,
        pltpu.SemaphoreType.DMA,
        pltpu.SemaphoreType.DMA,
        pltpu.SemaphoreType.DMA,
        pltpu.SemaphoreType.DMA,
    ],
    compiler_params=pltpu.CompilerParams(needs_layout_passes=False),
  )
  def _edge_score(hh_hbm, ht_hbm, rel_hbm, src_hbm, dst_hbm, typ_hbm, out_hbm,
                  src_v, dst_v, typ_v, hh0, hh1, ht0, ht1, rel_v, out_v,
                  tr_v, sem0, sem1, semi0, semi1, semi2, semi3):
    wid = lax.axis_index("s") * NC + lax.axis_index("c")
    base = wid * EPW
    pltpu.sync_copy(rel_hbm, rel_v)


    bufs = ((hh0, ht0, sem0), (hh1, ht1, sem1))
    isems = (semi0, semi1, semi2, semi3)

    def issue_idx(r, slot):
      off = base + r * B
      sem = isems[slot]
      pltpu.async_copy(src_hbm.at[pl.ds(off, B)], src_v.at[slot], sem)
      pltpu.async_copy(dst_hbm.at[pl.ds(off, B)], dst_v.at[slot], sem)
      pltpu.async_copy(typ_hbm.at[pl.ds(off, B)], typ_v.at[slot], sem)

    def wait_idx(r, slot):
      off = base + r * B
      sem = isems[slot]
      pltpu.make_async_copy(src_hbm.at[pl.ds(off, B)], src_v.at[slot], sem).wait()
      pltpu.make_async_copy(dst_hbm.at[pl.ds(off, B)], dst_v.at[slot], sem).wait()
      pltpu.make_async_copy(typ_hbm.at[pl.ds(off, B)], typ_v.at[slot], sem).wait()

    def issue(r, parity, slot):
      hh_b, ht_b, sem = bufs[parity]
      pltpu.async_copy(hh_hbm.at[src_v.at[slot]], hh_b, sem)
      pltpu.async_copy(ht_hbm.at[dst_v.at[slot]], ht_b, sem)

    def wait(r, parity, slot):
      hh_b, ht_b, sem = bufs[parity]
      pltpu.make_async_copy(hh_hbm.at[src_v.at[slot]], hh_b, sem).wait()
      pltpu.make_async_copy(ht_hbm.at[dst_v.at[slot]], ht_b, sem).wait()

    lanes = lax.iota(jnp.int32, L)

    def compute(r, parity, iparity):
      hh_b, ht_b, _ = bufs[parity]
      off = r * B

      @plsc.parallel_loop(0, B, unroll=3)
      def _(e):
        tvec = typ_v[iparity, pl.ds(e - (e & (L - 1)), L)]
        tsp = _vperm(tvec, jnp.full((L,), e & (L - 1), jnp.int32))
        acc = jnp.zeros((L,), jnp.float32)
        for j in range(KJ):
          sl = pl.ds(j * L, L)
          rl = plsc.load_gather(rel_v, [tsp, lanes + (j * L)])
          acc = acc + hh_b[e, sl] * rl * ht_b[e, sl]
        tr_v[e >> 4, e & (L - 1), pl.ds(0, L)] = acc

      @plsc.parallel_loop(0, B // L)
      def _(g):
        gg = jnp.full((L,), g, jnp.int32)
        sums = jnp.zeros((L,), jnp.float32)
        for k in range(L):
          sums = sums + plsc.load_gather(
              tr_v, [gg, lanes, jnp.full((L,), k, jnp.int32)])
        out_v[pl.ds(off + g * L, L)] = 1.0 / (1.0 + jnp.exp(-sums))

    issue_idx(0, 0)
    issue_idx(1, 1)
    issue_idx(2, 2)
    wait_idx(0, 0)
    issue(0, 0, 0)

    def step(k, q):
      wait(k, q % 2, q)
      wait_idx(k + 1, (q + 1) % 4)
      issue(k + 1, (q + 1) % 2, (q + 1) % 4)
      compute(k, q % 2, q)

      @pl.when(k + 3 < ROUNDS)
      def _():
        issue_idx(k + 3, (q + 3) % 4)

    def quad_round(i, carry):
      r = 4 * i
      for q in range(4):
        step(r + q, q)
      return carry

    lax.fori_loop(0, (ROUNDS - 1) // 4, quad_round, 0)
    wait(ROUNDS - 1, 0, 0)
    compute(ROUNDS - 1, 0, 0)
    pltpu.sync_copy(out_v, out_hbm.at[pl.ds(base, EPW)])

  return _edge_score


def kernel(x, edge_index, edge_index_neighborhood, edge_type,
           W_head, b_head, W_tail, b_tail, rel_table):
    del edge_index_neighborhood
    hh, ht = _node_transform(x, W_head.T, W_tail.T,
                             b_head.reshape(1, H), b_tail.reshape(1, H))
    src = edge_index[0]
    dst = edge_index[1]
    return _build_edge_score()(hh, ht, rel_table, src, dst, edge_type)


# speedup vs baseline: 1.0897x; 1.0897x over previous
import functools

import jax
import jax.numpy as jnp
from jax import lax
from jax.experimental import pallas as pl
from jax.experimental.pallas import tpu as pltpu
from jax.experimental.pallas import tpu_sc as plsc

N, E, D, H, R = 10000, 320000, 128, 128, 500

NC, NS, L = 2, 16, 16
NW = NC * NS
EPW = E // NW
B = 80
ROUNDS = EPW // B
KJ = H // L

M_BLK = 400

_DNUMS = lax.GatherDimensionNumbers(
    offset_dims=(), collapsed_slice_dims=(0,), start_index_map=(0,))


def _vperm(vec, idx):
  return lax.gather(vec, idx[:, None], _DNUMS, (1,),
                    mode=lax.GatherScatterMode.PROMISE_IN_BOUNDS)


def _transform_body(x_ref, wh_ref, wt_ref, bh_ref, bt_ref, hh_ref, ht_ref):
    xb = x_ref[...]
    hh_ref[...] = jnp.dot(xb, wh_ref[...],
                          preferred_element_type=jnp.float32) + bh_ref[...]
    ht_ref[...] = jnp.dot(xb, wt_ref[...],
                          preferred_element_type=jnp.float32) + bt_ref[...]


_node_transform = pl.pallas_call(
    _transform_body,
    grid=(N // M_BLK,),
    in_specs=[
        pl.BlockSpec((M_BLK, D), lambda i: (i, 0)),
        pl.BlockSpec((D, H), lambda i: (0, 0)),
        pl.BlockSpec((D, H), lambda i: (0, 0)),
        pl.BlockSpec((1, H), lambda i: (0, 0)),
        pl.BlockSpec((1, H), lambda i: (0, 0)),
    ],
    out_specs=[
        pl.BlockSpec((M_BLK, H), lambda i: (i, 0)),
        pl.BlockSpec((M_BLK, H), lambda i: (i, 0)),
    ],
    out_shape=[
        jax.ShapeDtypeStruct((N, H), jnp.float32),
        jax.ShapeDtypeStruct((N, H), jnp.float32),
    ],
)


@functools.cache
def _build_edge_score():
  @functools.partial(
    pl.kernel,
    out_type=jax.ShapeDtypeStruct((E,), jnp.float32),
    mesh=plsc.VectorSubcoreMesh(core_axis_name="c", subcore_axis_name="s",
                                num_cores=NC, num_subcores=NS),
    scratch_types=[
        pltpu.VMEM((4, B), jnp.int32),
        pltpu.VMEM((4, B), jnp.int32),
        pltpu.VMEM((4, B), jnp.int32),
        pltpu.VMEM((B, H), jnp.float32),
        pltpu.VMEM((B, H), jnp.float32),
        pltpu.VMEM((B, H), jnp.float32),
        pltpu.VMEM((B, H), jnp.float32),
        pltpu.VMEM((R, H), jnp.float32),
        pltpu.VMEM((EPW,), jnp.float32),
        pltpu.VMEM((B // L, L, 17), jnp.float32),
        pltpu.SemaphoreType.DMA,
        pltpu.SemaphoreType.DMA,
        pltpu.SemaphoreType.DMA,
        pltpu.SemaphoreType.DMA,
        pltpu.SemaphoreType.DMA,
        pltpu.SemaphoreType.DMA,
    ],
    compiler_params=pltpu.CompilerParams(needs_layout_passes=False),
  )
  def _edge_score(hh_hbm, ht_hbm, rel_hbm, src_hbm, dst_hbm, typ_hbm, out_hbm,
                  src_v, dst_v, typ_v, hh0, hh1, ht0, ht1, rel_v, out_v,
                  tr_v, sem0, sem1, semi0, semi1, semi2, semi3):
    wid = lax.axis_index("s") * NC + lax.axis_index("c")
    base = wid * EPW
    pltpu.sync_copy(rel_hbm, rel_v)


    bufs = ((hh0, ht0, sem0), (hh1, ht1, sem1))
    isems = (semi0, semi1, semi2, semi3)

    def issue_idx(r, slot):
      off = base + r * B
      sem = isems[slot]
      pltpu.async_copy(src_hbm.at[pl.ds(off, B)], src_v.at[slot], sem)
      pltpu.async_copy(dst_hbm.at[pl.ds(off, B)], dst_v.at[slot], sem)
      pltpu.async_copy(typ_hbm.at[pl.ds(off, B)], typ_v.at[slot], sem)

    def wait_idx(r, slot):
      off = base + r * B
      sem = isems[slot]
      pltpu.make_async_copy(src_hbm.at[pl.ds(off, B)], src_v.at[slot], sem).wait()
      pltpu.make_async_copy(dst_hbm.at[pl.ds(off, B)], dst_v.at[slot], sem).wait()
      pltpu.make_async_copy(typ_hbm.at[pl.ds(off, B)], typ_v.at[slot], sem).wait()

    def issue(r, parity, slot):
      hh_b, ht_b, sem = bufs[parity]
      pltpu.async_copy(hh_hbm.at[src_v.at[slot]], hh_b, sem)
      pltpu.async_copy(ht_hbm.at[dst_v.at[slot]], ht_b, sem)

    def wait(r, parity, slot):
      hh_b, ht_b, sem = bufs[parity]
      pltpu.make_async_copy(hh_hbm.at[src_v.at[slot]], hh_b, sem).wait()
      pltpu.make_async_copy(ht_hbm.at[dst_v.at[slot]], ht_b, sem).wait()

    lanes = lax.iota(jnp.int32, L)

    def compute(r, parity, iparity):
      hh_b, ht_b, _ = bufs[parity]
      off = r * B

      @plsc.parallel_loop(0, B, unroll=2)
      def _(e):
        tvec = typ_v[iparity, pl.ds(e - (e & (L - 1)), L)]
        tsp = _vperm(tvec, jnp.full((L,), e & (L - 1), jnp.int32))
        acc = jnp.zeros((L,), jnp.float32)
        for j in range(KJ):
          sl = pl.ds(j * L, L)
          rl = plsc.load_gather(rel_v, [tsp, lanes + (j * L)])
          acc = acc + hh_b[e, sl] * rl * ht_b[e, sl]
        tr_v[e >> 4, e & (L - 1), pl.ds(0, L)] = acc

      @plsc.parallel_loop(0, B // L)
      def _(g):
        gg = jnp.full((L,), g, jnp.int32)
        sums = jnp.zeros((L,), jnp.float32)
        for k in range(L):
          sums = sums + plsc.load_gather(
              tr_v, [gg, lanes, jnp.full((L,), k, jnp.int32)])
        out_v[pl.ds(off + g * L, L)] = 1.0 / (1.0 + jnp.exp(-sums))

    issue_idx(0, 0)
    issue_idx(1, 1)
    issue_idx(2, 2)
    wait_idx(0, 0)
    issue(0, 0, 0)

    def step(k, q):
      wait(k, q % 2, q)
      wait_idx(k + 1, (q + 1) % 4)
      issue(k + 1, (q + 1) % 2, (q + 1) % 4)
      compute(k, q % 2, q)

      @pl.when(k + 3 < ROUNDS)
      def _():
        issue_idx(k + 3, (q + 3) % 4)

    def quad_round(i, carry):
      r = 4 * i
      for q in range(4):
        step(r + q, q)
      return carry

    lax.fori_loop(0, (ROUNDS - 1) // 4, quad_round, 0)
    wait(ROUNDS - 1, 0, 0)
    compute(ROUNDS - 1, 0, 0)
    pltpu.sync_copy(out_v, out_hbm.at[pl.ds(base, EPW)])

  return _edge_score


def kernel(x, edge_index, edge_index_neighborhood, edge_type,
           W_head, b_head, W_tail, b_tail, rel_table):
    del edge_index_neighborhood
    hh, ht = _node_transform(x, W_head.T, W_tail.T,
                             b_head.reshape(1, H), b_tail.reshape(1, H))
    src = edge_index[0]
    dst = edge_index[1]
    return _build_edge_score()(hh, ht, rel_table, src, dst, edge_type)
